# baseline (speedup 1.0000x reference)
import functools

import jax
import jax.numpy as jnp
from jax import lax
from jax.experimental import pallas as pl
from jax.experimental.pallas import tpu as pltpu
from jax.experimental.pallas import tpu_sc as plsc

NSC = 4096
R_TC = 2048
CHUNK = 8
_NCORES = 2
_NSUB = 16
_NW = _NCORES * _NSUB


def _sc_body(x_hbm, t_hbm, a_hbm, w_hbm, s_hbm, xt_hbm, at_hbm,
             chunkbuf0, chunkbuf1, sem0, sem1, tbuf, abuf,
             wstg, sstg, xtstg, atstg):
    c = 1000
    rows_w = NSC // _NW
    nchunks = rows_w // CHUNK
    wid = lax.axis_index("s") * _NCORES + lax.axis_index("c")
    r0 = wid * rows_w

    pltpu.sync_copy(t_hbm.at[pl.ds(r0, rows_w)], tbuf.at[pl.ds(0, rows_w)])
    pltpu.sync_copy(a_hbm, abuf)

    lane = lax.broadcasted_iota(jnp.int32, (16,), 0)
    tail_mask = lane >= 8
    zero16 = jnp.zeros((16,), jnp.float32)
    full_offs = [16 * j for j in range(c // 16)]
    tail_off = c - 16
    bufs = (chunkbuf0, chunkbuf1)
    sems = (sem0, sem1)

    def copy_for(ci, b):
        return pltpu.make_async_copy(
            x_hbm.at[pl.ds(r0 + ci * CHUNK, CHUNK)], bufs[b], sems[b])

    def compute(ci, b):
        buf = bufs[b]
        t16 = tbuf[pl.ds(ci * CHUNK, 16)]
        for r in range(CHUNK):
            off16 = (ci * CHUNK + r) * 16
            ms = [buf[r, pl.ds(o, 16)] for o in full_offs[:4]]
            for j, o in enumerate(full_offs[4:]):
                ms[j % 4] = jnp.maximum(ms[j % 4], buf[r, pl.ds(o, 16)])
            ms[3] = jnp.maximum(ms[3], buf[r, pl.ds(tail_off, 16)])
            macc = jnp.maximum(jnp.maximum(ms[0], ms[1]),
                               jnp.maximum(ms[2], ms[3]))
            t_r = t16[r]
            zs = [zero16, zero16, zero16, zero16]
            xs = [zero16, zero16, zero16, zero16]
            for j, o in enumerate(full_offs):
                v = buf[r, pl.ds(o, 16)]
                zs[j % 4] = zs[j % 4] + jnp.exp(v - macc)
                xs[j % 4] = xs[j % 4] + jnp.where(lane == t_r - o, v, 0.0)
            vt = buf[r, pl.ds(tail_off, 16)]
            zs[0] = zs[0] + jnp.where(tail_mask, jnp.exp(vt - macc), 0.0)
            xs[0] = xs[0] + jnp.where(
                jnp.logical_and(tail_mask, lane == t_r - tail_off), vt, 0.0)
            zacc = (zs[0] + zs[1]) + (zs[2] + zs[3])
            xtacc = (xs[0] + xs[1]) + (xs[2] + xs[3])
            t_al = (t_r // 16) * 16
            hit = lane == (t_r - t_al)
            wstg[pl.ds(off16, 16)] = macc
            sstg[pl.ds(off16, 16)] = zacc
            xtstg[pl.ds(off16, 16)] = xtacc
            atstg[pl.ds(off16, 16)] = jnp.where(hit, abuf[pl.ds(t_al, 16)],
                                                0.0)

    copy_for(0, 0).start()

    def pair_body(cio, _):
        ci0 = cio * 2
        copy_for(ci0 + 1, 1).start()
        copy_for(ci0, 0).wait()
        compute(ci0, 0)

        @pl.when(ci0 + 2 < nchunks)
        def _():
            copy_for(ci0 + 2, 0).start()

        copy_for(ci0 + 1, 1).wait()
        compute(ci0 + 1, 1)
        return 0

    lax.fori_loop(0, nchunks // 2, pair_body, 0)

    pltpu.sync_copy(wstg, w_hbm.at[pl.ds(r0 * 16, rows_w * 16)])
    pltpu.sync_copy(sstg, s_hbm.at[pl.ds(r0 * 16, rows_w * 16)])
    pltpu.sync_copy(xtstg, xt_hbm.at[pl.ds(r0 * 16, rows_w * 16)])
    pltpu.sync_copy(atstg, at_hbm.at[pl.ds(r0 * 16, rows_w * 16)])


def _sc_stats(inputs, targets_sc, alpha_pad):
    c = inputs.shape[1]
    rows_w = NSC // _NW
    mesh = plsc.VectorSubcoreMesh(
        core_axis_name="c", subcore_axis_name="s",
        num_cores=_NCORES, num_subcores=_NSUB)
    f32 = jnp.float32
    run = pl.kernel(
        _sc_body,
        out_type=[jax.ShapeDtypeStruct((NSC * 16,), f32)] * 4,
        mesh=mesh,
        scratch_types=[
            pltpu.VMEM((CHUNK, c), f32),
            pltpu.VMEM((CHUNK, c), f32),
            pltpu.SemaphoreType.DMA,
            pltpu.SemaphoreType.DMA,
            pltpu.VMEM((rows_w + 16, ), jnp.int32),
            pltpu.VMEM((1024,), f32),
            pltpu.VMEM((rows_w * 16,), f32),
            pltpu.VMEM((rows_w * 16,), f32),
            pltpu.VMEM((rows_w * 16,), f32),
            pltpu.VMEM((rows_w * 16,), f32),
        ],
    )
    return run(inputs, targets_sc, alpha_pad)


def _tc_body(x_ref, t_ref, a_ref, o_ref, *, n_total):
    i = pl.program_id(0)
    x = x_ref[...]
    t = t_ref[...]
    r, c = x.shape
    m = jnp.max(x, axis=1, keepdims=True)
    z = jnp.sum(jnp.exp(x - m), axis=1, keepdims=True)
    col = lax.broadcasted_iota(jnp.int32, (r, c), 1)
    msk = col == t
    xt = jnp.sum(jnp.where(msk, x, 0.0), axis=1, keepdims=True)
    at = jnp.sum(jnp.where(msk, a_ref[...], 0.0), axis=1, keepdims=True)
    logp = (xt - m) - jnp.log(z)
    p = jnp.exp(logp)
    q = 1.0 - p
    loss = -at * q * q * logp
    s = jnp.sum(loss, keepdims=True).reshape(1, 1) * (1.0 / n_total)

    @pl.when(i == 0)
    def _():
        o_ref[...] = jnp.zeros_like(o_ref)

    o_ref[...] += s


def _tc_partial(inputs, targets_tc, alpha, n_total):
    n, c = inputs.shape
    nb = (n - NSC) // R_TC
    off = NSC // R_TC
    t2 = targets_tc.reshape(n - NSC, 1)
    a_row = alpha.reshape(1, c)
    return pl.pallas_call(
        functools.partial(_tc_body, n_total=n_total),
        grid=(nb,),
        in_specs=[
            pl.BlockSpec((R_TC, c), lambda i: (i + off, 0)),
            pl.BlockSpec((R_TC, 1), lambda i: (i, 0)),
            pl.BlockSpec((1, c), lambda i: (0, 0)),
        ],
        out_specs=pl.BlockSpec((1, 1), lambda i: (0, 0)),
        out_shape=jax.ShapeDtypeStruct((1, 1), jnp.float32),
    )(inputs, t2, a_row)


def _combine_body(w_ref, s_ref, xt_ref, at_ref, tc_ref, o_ref, *, n_total):
    w = w_ref[...]
    s = s_ref[...]
    g, l = w.shape
    lane = lax.broadcasted_iota(jnp.int32, (g, l), 1)
    y = w
    for sh in (1, 2, 4, 8):
        y = jnp.maximum(y, pltpu.roll(y, l - sh, 1))
    y = jnp.where(lane % 16 == 0, y, -jnp.inf)
    for sh in (1, 2, 4, 8):
        y = jnp.maximum(y, pltpu.roll(y, sh, 1))
    m = y
    ri = lax.broadcasted_iota(jnp.int32, (128, 128), 0)
    ci = lax.broadcasted_iota(jnp.int32, (128, 128), 1)
    sel = (ri // 16 == ci // 16).astype(jnp.float32)
    e = jnp.exp(w - m) * s
    z = jax.lax.dot(e, sel, precision=jax.lax.Precision.HIGHEST)
    xt = jax.lax.dot(xt_ref[...], sel, precision=jax.lax.Precision.HIGHEST)
    at = jax.lax.dot(at_ref[...], sel, precision=jax.lax.Precision.HIGHEST)
    logp = (xt - m) - jnp.log(z)
    p = jnp.exp(logp)
    q = 1.0 - p
    loss = -at * q * q * logp
    ssum = jnp.sum(loss, keepdims=True).reshape(1, 1) * (1.0 / (16 * n_total))
    o_ref[...] = tc_ref[...] + ssum


def _combine(w_sc, s_sc, xt_sc, at_sc, tc_part, n_total):
    gdim = NSC * 16 // 128
    w2 = w_sc.reshape(gdim, 128)
    s2 = s_sc.reshape(gdim, 128)
    xt2 = xt_sc.reshape(gdim, 128)
    at2 = at_sc.reshape(gdim, 128)
    return pl.pallas_call(
        functools.partial(_combine_body, n_total=n_total),
        out_shape=jax.ShapeDtypeStruct((1, 1), jnp.float32),
    )(w2, s2, xt2, at2, tc_part)


def kernel(inputs, targets, alpha):
    n, c = inputs.shape
    alpha_pad = jnp.zeros((1024,), jnp.float32).at[:c].set(alpha.reshape(c))
    tc_part = _tc_partial(inputs, targets[NSC:], alpha, n)
    w_sc, s_sc, xt_sc, at_sc = _sc_stats(inputs, targets[:NSC], alpha_pad)
    out = _combine(w_sc, s_sc, xt_sc, at_sc, tc_part, n)
    return out[0, 0]


# speedup vs baseline: 1.3009x; 1.3009x over previous
import functools

import jax
import jax.numpy as jnp
from jax import lax
from jax.experimental import pallas as pl
from jax.experimental.pallas import tpu as pltpu
from jax.experimental.pallas import tpu_sc as plsc

NSC = 1024
R_TC = 1024
CHUNK = 8
_NCORES = 2
_NSUB = 16
_NW = _NCORES * _NSUB


def _sc_body(x_hbm, t_hbm, a_hbm, w_hbm, s_hbm, xt_hbm, at_hbm,
             chunkbuf0, chunkbuf1, sem0, sem1, tbuf, abuf,
             wstg, sstg, xtstg, atstg):
    c = 1000
    rows_w = NSC // _NW
    nchunks = rows_w // CHUNK
    wid = lax.axis_index("s") * _NCORES + lax.axis_index("c")
    r0 = wid * rows_w

    pltpu.sync_copy(t_hbm.at[pl.ds(r0, rows_w)], tbuf.at[pl.ds(0, rows_w)])
    pltpu.sync_copy(a_hbm, abuf)

    lane = lax.broadcasted_iota(jnp.int32, (16,), 0)
    tail_mask = lane >= 8
    zero16 = jnp.zeros((16,), jnp.float32)
    full_offs = [16 * j for j in range(c // 16)]
    tail_off = c - 16
    bufs = (chunkbuf0, chunkbuf1)
    sems = (sem0, sem1)

    def copy_for(ci, b):
        return pltpu.make_async_copy(
            x_hbm.at[pl.ds(r0 + ci * CHUNK, CHUNK)], bufs[b], sems[b])

    def compute(ci, b):
        buf = bufs[b]
        t16 = tbuf[pl.ds(ci * CHUNK, 16)]
        for r in range(CHUNK):
            off16 = (ci * CHUNK + r) * 16
            ms = [buf[r, pl.ds(o, 16)] for o in full_offs[:4]]
            for j, o in enumerate(full_offs[4:]):
                ms[j % 4] = jnp.maximum(ms[j % 4], buf[r, pl.ds(o, 16)])
            ms[3] = jnp.maximum(ms[3], buf[r, pl.ds(tail_off, 16)])
            macc = jnp.maximum(jnp.maximum(ms[0], ms[1]),
                               jnp.maximum(ms[2], ms[3]))
            t_r = t16[r]
            zs = [zero16, zero16, zero16, zero16]
            xs = [zero16, zero16, zero16, zero16]
            for j, o in enumerate(full_offs):
                v = buf[r, pl.ds(o, 16)]
                zs[j % 4] = zs[j % 4] + jnp.exp(v - macc)
                xs[j % 4] = xs[j % 4] + jnp.where(lane == t_r - o, v, 0.0)
            vt = buf[r, pl.ds(tail_off, 16)]
            zs[0] = zs[0] + jnp.where(tail_mask, jnp.exp(vt - macc), 0.0)
            xs[0] = xs[0] + jnp.where(
                jnp.logical_and(tail_mask, lane == t_r - tail_off), vt, 0.0)
            zacc = (zs[0] + zs[1]) + (zs[2] + zs[3])
            xtacc = (xs[0] + xs[1]) + (xs[2] + xs[3])
            t_al = (t_r // 16) * 16
            hit = lane == (t_r - t_al)
            wstg[pl.ds(off16, 16)] = macc
            sstg[pl.ds(off16, 16)] = zacc
            xtstg[pl.ds(off16, 16)] = xtacc
            atstg[pl.ds(off16, 16)] = jnp.where(hit, abuf[pl.ds(t_al, 16)],
                                                0.0)

    copy_for(0, 0).start()

    def pair_body(cio, _):
        ci0 = cio * 2
        copy_for(ci0 + 1, 1).start()
        copy_for(ci0, 0).wait()
        compute(ci0, 0)

        @pl.when(ci0 + 2 < nchunks)
        def _():
            copy_for(ci0 + 2, 0).start()

        copy_for(ci0 + 1, 1).wait()
        compute(ci0 + 1, 1)
        return 0

    lax.fori_loop(0, nchunks // 2, pair_body, 0)

    pltpu.sync_copy(wstg, w_hbm.at[pl.ds(r0 * 16, rows_w * 16)])
    pltpu.sync_copy(sstg, s_hbm.at[pl.ds(r0 * 16, rows_w * 16)])
    pltpu.sync_copy(xtstg, xt_hbm.at[pl.ds(r0 * 16, rows_w * 16)])
    pltpu.sync_copy(atstg, at_hbm.at[pl.ds(r0 * 16, rows_w * 16)])


def _sc_stats(inputs, targets_sc, alpha_pad):
    c = inputs.shape[1]
    rows_w = NSC // _NW
    mesh = plsc.VectorSubcoreMesh(
        core_axis_name="c", subcore_axis_name="s",
        num_cores=_NCORES, num_subcores=_NSUB)
    f32 = jnp.float32
    run = pl.kernel(
        _sc_body,
        out_type=[jax.ShapeDtypeStruct((NSC * 16,), f32)] * 4,
        mesh=mesh,
        scratch_types=[
            pltpu.VMEM((CHUNK, c), f32),
            pltpu.VMEM((CHUNK, c), f32),
            pltpu.SemaphoreType.DMA,
            pltpu.SemaphoreType.DMA,
            pltpu.VMEM((rows_w + 16, ), jnp.int32),
            pltpu.VMEM((1024,), f32),
            pltpu.VMEM((rows_w * 16,), f32),
            pltpu.VMEM((rows_w * 16,), f32),
            pltpu.VMEM((rows_w * 16,), f32),
            pltpu.VMEM((rows_w * 16,), f32),
        ],
    )
    return run(inputs, targets_sc, alpha_pad)


def _tc_body(x_ref, t_ref, a_ref, o_ref, *, n_total):
    i = pl.program_id(0)
    x = x_ref[...]
    t = t_ref[...]
    r, c = x.shape
    m = jnp.max(x, axis=1, keepdims=True)
    z = jnp.sum(jnp.exp(x - m), axis=1, keepdims=True)
    col = lax.broadcasted_iota(jnp.int32, (r, c), 1)
    msk = col == t
    xt = jnp.sum(jnp.where(msk, x, 0.0), axis=1, keepdims=True)
    at = jnp.sum(jnp.where(msk, a_ref[...], 0.0), axis=1, keepdims=True)
    logp = (xt - m) - jnp.log(z)
    p = jnp.exp(logp)
    q = 1.0 - p
    loss = -at * q * q * logp
    s = jnp.sum(loss, keepdims=True).reshape(1, 1) * (1.0 / n_total)

    @pl.when(i == 0)
    def _():
        o_ref[...] = jnp.zeros_like(o_ref)

    o_ref[...] += s


def _tc_partial(inputs, targets_tc, alpha, n_total):
    n, c = inputs.shape
    nb = (n - NSC) // R_TC
    off = NSC // R_TC
    t2 = targets_tc.reshape(n - NSC, 1)
    a_row = alpha.reshape(1, c)
    return pl.pallas_call(
        functools.partial(_tc_body, n_total=n_total),
        grid=(nb,),
        in_specs=[
            pl.BlockSpec((R_TC, c), lambda i: (i + off, 0)),
            pl.BlockSpec((R_TC, 1), lambda i: (i, 0)),
            pl.BlockSpec((1, c), lambda i: (0, 0)),
        ],
        out_specs=pl.BlockSpec((1, 1), lambda i: (0, 0)),
        out_shape=jax.ShapeDtypeStruct((1, 1), jnp.float32),
    )(inputs, t2, a_row)


def _combine_body(w_ref, s_ref, xt_ref, at_ref, tc_ref, o_ref, *, n_total):
    w = w_ref[...]
    s = s_ref[...]
    g, l = w.shape
    lane = lax.broadcasted_iota(jnp.int32, (g, l), 1)
    y = w
    for sh in (1, 2, 4, 8):
        y = jnp.maximum(y, pltpu.roll(y, l - sh, 1))
    y = jnp.where(lane % 16 == 0, y, -jnp.inf)
    for sh in (1, 2, 4, 8):
        y = jnp.maximum(y, pltpu.roll(y, sh, 1))
    m = y
    ri = lax.broadcasted_iota(jnp.int32, (128, 128), 0)
    ci = lax.broadcasted_iota(jnp.int32, (128, 128), 1)
    sel = (ri // 16 == ci // 16).astype(jnp.float32)
    e = jnp.exp(w - m) * s
    z = jax.lax.dot(e, sel, precision=jax.lax.Precision.HIGHEST)
    xt = jax.lax.dot(xt_ref[...], sel, precision=jax.lax.Precision.HIGHEST)
    at = jax.lax.dot(at_ref[...], sel, precision=jax.lax.Precision.HIGHEST)
    logp = (xt - m) - jnp.log(z)
    p = jnp.exp(logp)
    q = 1.0 - p
    loss = -at * q * q * logp
    ssum = jnp.sum(loss, keepdims=True).reshape(1, 1) * (1.0 / (16 * n_total))
    o_ref[...] = tc_ref[...] + ssum


def _combine(w_sc, s_sc, xt_sc, at_sc, tc_part, n_total):
    gdim = NSC * 16 // 128
    w2 = w_sc.reshape(gdim, 128)
    s2 = s_sc.reshape(gdim, 128)
    xt2 = xt_sc.reshape(gdim, 128)
    at2 = at_sc.reshape(gdim, 128)
    return pl.pallas_call(
        functools.partial(_combine_body, n_total=n_total),
        out_shape=jax.ShapeDtypeStruct((1, 1), jnp.float32),
    )(w2, s2, xt2, at2, tc_part)


def kernel(inputs, targets, alpha):
    n, c = inputs.shape
    alpha_pad = jnp.zeros((1024,), jnp.float32).at[:c].set(alpha.reshape(c))
    tc_part = _tc_partial(inputs, targets[NSC:], alpha, n)
    w_sc, s_sc, xt_sc, at_sc = _sc_stats(inputs, targets[:NSC], alpha_pad)
    out = _combine(w_sc, s_sc, xt_sc, at_sc, tc_part, n)
    return out[0, 0]
